# baseline (speedup 1.0000x reference)
import functools

import jax
import jax.numpy as jnp
from jax import lax
from jax.experimental import pallas as pl
from jax.experimental.pallas import tpu as pltpu
from jax.experimental.pallas import tpu_sc as plsc

U = 100000
B = 16384
D = 64
H1 = 64
H2 = 32
EPS = 1e-5

NC = 2
NS = 16
NW = NC * NS
FPW = 4 * D // NW
QTR = B // 4

BKC = 2048


def _sc_gather(users, items, umf_t, imf_t, umlp_t, imlp_t):
    mesh = plsc.VectorSubcoreMesh(core_axis_name="c", subcore_axis_name="s")

    @functools.partial(
        pl.kernel,
        mesh=mesh,
        compiler_params=pltpu.CompilerParams(needs_layout_passes=False),
        out_type=jax.ShapeDtypeStruct((4 * D, B), jnp.float32),
        scratch_types=[
            pltpu.VMEM((U,), jnp.float32),
            pltpu.VMEM((B,), jnp.int32),
            pltpu.VMEM((2, QTR), jnp.float32),
            pltpu.SemaphoreType.DMA,
        ],
    )
    def sc_kernel(users_h, items_h, umf_h, imf_h, umlp_h, imlp_h,
                  out_o, colbuf, idx_v, outq, sem):
        wid = lax.axis_index("s") * NC + lax.axis_index("c")
        f0 = wid * 2
        pairs = [(users_h, [(0, umf_h), (2, umlp_h)]),
                 (items_h, [(1, imf_h), (3, imlp_h)])]
        for idx_h, tbls in pairs:
            pltpu.sync_copy(idx_h, idx_v)
            for t, tbl in tbls:
                for f in range(2):
                    col = f0 + f
                    pltpu.sync_copy(tbl.at[col], colbuf)
                    pend = [None, None]
                    for q in range(4):
                        slot = q % 2
                        if pend[slot] is not None:
                            pend[slot].wait()

                        def gather_body(v, carry, q=q, slot=slot):
                            base = q * QTR + v * 128
                            for k in range(8):
                                iv = idx_v[pl.ds(base + k * 16, 16)]
                                outq[slot, pl.ds(v * 128 + k * 16, 16)] = (
                                    plsc.load_gather(colbuf, [iv]))
                            return carry

                        lax.fori_loop(0, QTR // 128, gather_body, 0)
                        pend[slot] = pltpu.async_copy(
                            outq.at[slot],
                            out_o.at[t * D + col, pl.ds(q * QTR, QTR)],
                            sem)
                    for cp in pend:
                        if cp is not None:
                            cp.wait()

    return sc_kernel(users, items, umf_t, imf_t, umlp_t, imlp_t)


def _tc_body(g_r, w1at_r, w1bt_r, b1_r, w2ft_r, b2f_r,
             wmf_r, wmlp_r, c0_r, out_r):
    g = g_r[:]
    umf_g = g[0:D]
    imf_g = g[D:2 * D]
    ug_g = g[2 * D:3 * D]
    ig_g = g[3 * D:4 * D]
    h1 = jnp.dot(w1at_r[:], ug_g, preferred_element_type=jnp.float32)
    h1 = h1 + jnp.dot(w1bt_r[:], ig_g, preferred_element_type=jnp.float32)
    h1 = jnp.maximum(h1 + b1_r[:], 0.0)
    h2 = jnp.dot(w2ft_r[:], h1, preferred_element_type=jnp.float32) + b2f_r[:]
    h2 = jnp.maximum(h2, 0.0)
    prod = umf_g * imf_g
    mf = jnp.dot(wmf_r[:], prod, preferred_element_type=jnp.float32)
    ml = jnp.dot(wmlp_r[:], h2, preferred_element_type=jnp.float32)
    out_r[:] = mf + ml + c0_r[0, 0]


def _tc_head(g, w1at, w1bt, b1c, w2ft, b2fc, wmf_row, wmlp_row, c0):
    def bs_full(shape):
        return pl.BlockSpec(shape, lambda i: (0,) * len(shape))

    return pl.pallas_call(
        _tc_body,
        grid=(B // BKC,),
        in_specs=[
            pl.BlockSpec((4 * D, BKC), lambda i: (0, i)),
            bs_full((D, H1)), bs_full((D, H1)), bs_full((H1, 1)),
            bs_full((H2, H1)), bs_full((H2, 1)),
            bs_full((1, D)), bs_full((1, H2)), bs_full((1, 1)),
        ],
        out_specs=pl.BlockSpec((1, BKC), lambda i: (0, i)),
        out_shape=jax.ShapeDtypeStruct((1, B), jnp.float32),
    )(g, w1at, w1bt, b1c, w2ft, b2fc, wmf_row, wmlp_row, c0)


def kernel(users, items, user_mf, item_mf, user_mlp, item_mlp,
           W1, b1, g1, be1, m1, v1, W2, b2, g2, be2, m2, v2, Wp, bp):
    users = users.astype(jnp.int32)
    items = items.astype(jnp.int32)

    g = _sc_gather(users, items,
                   user_mf.T, item_mf.T, user_mlp.T, item_mlp.T)

    s1 = g1 / jnp.sqrt(v1 + EPS)
    t1 = be1 - m1 * s1
    s2 = g2 / jnp.sqrt(v2 + EPS)
    t2 = be2 - m2 * s2
    w1at = W1[:D].T
    w1bt = W1[D:].T
    w2ft = (s1[:, None] * W2).T
    b2f = b2 + t1 @ W2
    wmf = Wp[:D, 0]
    wmlp = s2 * Wp[D:, 0]
    c0 = t2 @ Wp[D:, 0] + bp[0]

    out = _tc_head(g, w1at, w1bt, b1.reshape(H1, 1),
                   w2ft, b2f.reshape(H2, 1),
                   wmf.reshape(1, D), wmlp.reshape(1, H2),
                   c0.reshape(1, 1))
    return out[0]


# speedup vs baseline: 1.3197x; 1.3197x over previous
import functools

import jax
import jax.numpy as jnp
from jax import lax
from jax.experimental import pallas as pl
from jax.experimental.pallas import tpu as pltpu
from jax.experimental.pallas import tpu_sc as plsc

U = 100000
B = 16384
D = 64
H1 = 64
H2 = 32
EPS = 1e-5

NC = 2
NS = 16
NW = NC * NS
FPW = 4 * D // NW
QTR = B // 4

BKC = 2048


def _sc_gather(users, items, umf_t, imf_t, umlp_t, imlp_t):
    mesh = plsc.VectorSubcoreMesh(core_axis_name="c", subcore_axis_name="s")

    @functools.partial(
        pl.kernel,
        mesh=mesh,
        compiler_params=pltpu.CompilerParams(needs_layout_passes=False),
        out_type=jax.ShapeDtypeStruct((4 * D, B), jnp.float32),
        scratch_types=[
            pltpu.VMEM((U,), jnp.float32),
            pltpu.VMEM((B,), jnp.int32),
            pltpu.VMEM((B // 2,), jnp.float32),
        ],
    )
    def sc_kernel(users_h, items_h, umf_h, imf_h, umlp_h, imlp_h,
                  out_o, colbuf, idx_v, outq):
        wid = lax.axis_index("s") * NC + lax.axis_index("c")
        f0 = wid * 2
        pairs = [(users_h, [(0, umf_h), (2, umlp_h)]),
                 (items_h, [(1, imf_h), (3, imlp_h)])]
        for idx_h, tbls in pairs:
            pltpu.sync_copy(idx_h, idx_v)
            for t, tbl in tbls:
                for f in range(2):
                    col = f0 + f
                    pltpu.sync_copy(tbl.at[col], colbuf)
                    for half in range(2):

                        def gather_body(v, carry, half=half):
                            base = half * (B // 2) + v * 128
                            for k in range(8):
                                iv = idx_v[pl.ds(base + k * 16, 16)]
                                outq[pl.ds(v * 128 + k * 16, 16)] = (
                                    plsc.load_gather(colbuf, [iv]))
                            return carry

                        lax.fori_loop(0, (B // 2) // 128, gather_body, 0)
                        pltpu.sync_copy(
                            outq,
                            out_o.at[t * D + col,
                                     pl.ds(half * (B // 2), B // 2)])

    return sc_kernel(users, items, umf_t, imf_t, umlp_t, imlp_t)


def _tc_body(g_r, w1at_r, w1bt_r, b1_r, w2ft_r, b2f_r,
             wmf_r, wmlp_r, c0_r, out_r):
    g = g_r[:]
    umf_g = g[0:D]
    imf_g = g[D:2 * D]
    ug_g = g[2 * D:3 * D]
    ig_g = g[3 * D:4 * D]
    h1 = jnp.dot(w1at_r[:], ug_g, preferred_element_type=jnp.float32)
    h1 = h1 + jnp.dot(w1bt_r[:], ig_g, preferred_element_type=jnp.float32)
    h1 = jnp.maximum(h1 + b1_r[:], 0.0)
    h2 = jnp.dot(w2ft_r[:], h1, preferred_element_type=jnp.float32) + b2f_r[:]
    h2 = jnp.maximum(h2, 0.0)
    prod = umf_g * imf_g
    mf = jnp.dot(wmf_r[:], prod, preferred_element_type=jnp.float32)
    ml = jnp.dot(wmlp_r[:], h2, preferred_element_type=jnp.float32)
    out_r[:] = mf + ml + c0_r[0, 0]


def _tc_head(g, w1at, w1bt, b1c, w2ft, b2fc, wmf_row, wmlp_row, c0):
    def bs_full(shape):
        return pl.BlockSpec(shape, lambda i: (0,) * len(shape))

    return pl.pallas_call(
        _tc_body,
        grid=(B // BKC,),
        in_specs=[
            pl.BlockSpec((4 * D, BKC), lambda i: (0, i)),
            bs_full((D, H1)), bs_full((D, H1)), bs_full((H1, 1)),
            bs_full((H2, H1)), bs_full((H2, 1)),
            bs_full((1, D)), bs_full((1, H2)), bs_full((1, 1)),
        ],
        out_specs=pl.BlockSpec((1, BKC), lambda i: (0, i)),
        out_shape=jax.ShapeDtypeStruct((1, B), jnp.float32),
    )(g, w1at, w1bt, b1c, w2ft, b2fc, wmf_row, wmlp_row, c0)


def kernel(users, items, user_mf, item_mf, user_mlp, item_mlp,
           W1, b1, g1, be1, m1, v1, W2, b2, g2, be2, m2, v2, Wp, bp):
    users = users.astype(jnp.int32)
    items = items.astype(jnp.int32)

    g = _sc_gather(users, items,
                   user_mf.T, item_mf.T, user_mlp.T, item_mlp.T)

    s1 = g1 / jnp.sqrt(v1 + EPS)
    t1 = be1 - m1 * s1
    s2 = g2 / jnp.sqrt(v2 + EPS)
    t2 = be2 - m2 * s2
    w1at = W1[:D].T
    w1bt = W1[D:].T
    w2ft = (s1[:, None] * W2).T
    b2f = b2 + t1 @ W2
    wmf = Wp[:D, 0]
    wmlp = s2 * Wp[D:, 0]
    c0 = t2 @ Wp[D:, 0] + bp[0]

    out = _tc_head(g, w1at, w1bt, b1.reshape(H1, 1),
                   w2ft, b2f.reshape(H2, 1),
                   wmf.reshape(1, D), wmlp.reshape(1, H2),
                   c0.reshape(1, 1))
    return out[0]


# speedup vs baseline: 1.3886x; 1.0522x over previous
import functools

import jax
import jax.numpy as jnp
from jax import lax
from jax.experimental import pallas as pl
from jax.experimental.pallas import tpu as pltpu
from jax.experimental.pallas import tpu_sc as plsc

U = 100000
B = 16384
D = 64
H1 = 64
H2 = 32
EPS = 1e-5

NC = 2
NS = 16
NW = NC * NS
FPW = 4 * D // NW
QTR = B // 4

BKC = 4096


def _sc_gather(users, items, umf_t, imf_t, umlp_t, imlp_t):
    mesh = plsc.VectorSubcoreMesh(core_axis_name="c", subcore_axis_name="s")

    @functools.partial(
        pl.kernel,
        mesh=mesh,
        compiler_params=pltpu.CompilerParams(needs_layout_passes=False),
        out_type=jax.ShapeDtypeStruct((4 * D, B), jnp.float32),
        scratch_types=[
            pltpu.VMEM((U,), jnp.float32),
            pltpu.VMEM((B,), jnp.int32),
            pltpu.VMEM((QTR,), jnp.float32),
            pltpu.VMEM((QTR,), jnp.float32),
            pltpu.SemaphoreType.DMA,
            pltpu.SemaphoreType.DMA,
        ],
    )
    def sc_kernel(users_h, items_h, umf_h, imf_h, umlp_h, imlp_h,
                  out_o, colbuf, idx_v, outq_a, outq_b, sem_a, sem_b):
        wid = lax.axis_index("s") * NC + lax.axis_index("c")
        f0 = wid * 2
        pairs = [(users_h, [(0, umf_h), (2, umlp_h)]),
                 (items_h, [(1, imf_h), (3, imlp_h)])]
        pend = [None, None]
        for idx_h, tbls in pairs:
            pltpu.sync_copy(idx_h, idx_v)
            for t, tbl in tbls:
                for f in range(2):
                    col = f0 + f
                    pltpu.sync_copy(tbl.at[col], colbuf)
                    for q in range(4):
                        slot = q % 2
                        outq = outq_a if slot == 0 else outq_b
                        sem = sem_a if slot == 0 else sem_b
                        if pend[slot] is not None:
                            pend[slot].wait()
                            pend[slot] = None

                        def gather_body(v, carry, q=q, outq=outq):
                            base = q * QTR + v * 128
                            for k in range(8):
                                iv = idx_v[pl.ds(base + k * 16, 16)]
                                outq[pl.ds(v * 128 + k * 16, 16)] = (
                                    plsc.load_gather(colbuf, [iv]))
                            return carry

                        lax.fori_loop(0, QTR // 128, gather_body, 0)
                        pend[slot] = pltpu.async_copy(
                            outq,
                            out_o.at[t * D + col, pl.ds(q * QTR, QTR)],
                            sem)
        for slot in range(2):
            if pend[slot] is not None:
                pend[slot].wait()

    return sc_kernel(users, items, umf_t, imf_t, umlp_t, imlp_t)


def _tc_body(g_r, w1at_r, w1bt_r, b1_r, w2ft_r, b2f_r,
             wmf_r, wmlp_r, c0_r, out_r):
    g = g_r[:]
    umf_g = g[0:D]
    imf_g = g[D:2 * D]
    ug_g = g[2 * D:3 * D]
    ig_g = g[3 * D:4 * D]
    h1 = jnp.dot(w1at_r[:], ug_g, preferred_element_type=jnp.float32)
    h1 = h1 + jnp.dot(w1bt_r[:], ig_g, preferred_element_type=jnp.float32)
    h1 = jnp.maximum(h1 + b1_r[:], 0.0)
    h2 = jnp.dot(w2ft_r[:], h1, preferred_element_type=jnp.float32) + b2f_r[:]
    h2 = jnp.maximum(h2, 0.0)
    prod = umf_g * imf_g
    mf = jnp.dot(wmf_r[:], prod, preferred_element_type=jnp.float32)
    ml = jnp.dot(wmlp_r[:], h2, preferred_element_type=jnp.float32)
    out_r[:] = mf + ml + c0_r[0, 0]


def _tc_head(g, w1at, w1bt, b1c, w2ft, b2fc, wmf_row, wmlp_row, c0):
    def bs_full(shape):
        return pl.BlockSpec(shape, lambda i: (0,) * len(shape))

    return pl.pallas_call(
        _tc_body,
        grid=(B // BKC,),
        in_specs=[
            pl.BlockSpec((4 * D, BKC), lambda i: (0, i)),
            bs_full((D, H1)), bs_full((D, H1)), bs_full((H1, 1)),
            bs_full((H2, H1)), bs_full((H2, 1)),
            bs_full((1, D)), bs_full((1, H2)), bs_full((1, 1)),
        ],
        out_specs=pl.BlockSpec((1, BKC), lambda i: (0, i)),
        out_shape=jax.ShapeDtypeStruct((1, B), jnp.float32),
    )(g, w1at, w1bt, b1c, w2ft, b2fc, wmf_row, wmlp_row, c0)


def kernel(users, items, user_mf, item_mf, user_mlp, item_mlp,
           W1, b1, g1, be1, m1, v1, W2, b2, g2, be2, m2, v2, Wp, bp):
    users = users.astype(jnp.int32)
    items = items.astype(jnp.int32)

    g = _sc_gather(users, items,
                   user_mf.T, item_mf.T, user_mlp.T, item_mlp.T)

    s1 = g1 / jnp.sqrt(v1 + EPS)
    t1 = be1 - m1 * s1
    s2 = g2 / jnp.sqrt(v2 + EPS)
    t2 = be2 - m2 * s2
    w1at = W1[:D].T
    w1bt = W1[D:].T
    w2ft = (s1[:, None] * W2).T
    b2f = b2 + t1 @ W2
    wmf = Wp[:D, 0]
    wmlp = s2 * Wp[D:, 0]
    c0 = t2 @ Wp[D:, 0] + bp[0]

    out = _tc_head(g, w1at, w1bt, b1.reshape(H1, 1),
                   w2ft, b2f.reshape(H2, 1),
                   wmf.reshape(1, D), wmlp.reshape(1, H2),
                   c0.reshape(1, 1))
    return out[0]


# speedup vs baseline: 1.3910x; 1.0017x over previous
import functools

import jax
import jax.numpy as jnp
from jax import lax
from jax.experimental import pallas as pl
from jax.experimental.pallas import tpu as pltpu
from jax.experimental.pallas import tpu_sc as plsc

U = 100000
B = 16384
D = 64
H1 = 64
H2 = 32
EPS = 1e-5

NC = 2
NS = 16
NW = NC * NS
FPW = 4 * D // NW
QTR = B // 4

BKC = 8192


def _sc_gather(users, items, umf_t, imf_t, umlp_t, imlp_t):
    mesh = plsc.VectorSubcoreMesh(core_axis_name="c", subcore_axis_name="s")

    @functools.partial(
        pl.kernel,
        mesh=mesh,
        compiler_params=pltpu.CompilerParams(needs_layout_passes=False),
        out_type=jax.ShapeDtypeStruct((4 * D, B), jnp.float32),
        scratch_types=[
            pltpu.VMEM((U,), jnp.float32),
            pltpu.VMEM((B,), jnp.int32),
            pltpu.VMEM((QTR,), jnp.float32),
            pltpu.VMEM((QTR,), jnp.float32),
            pltpu.SemaphoreType.DMA,
            pltpu.SemaphoreType.DMA,
        ],
    )
    def sc_kernel(users_h, items_h, umf_h, imf_h, umlp_h, imlp_h,
                  out_o, colbuf, idx_v, outq_a, outq_b, sem_a, sem_b):
        wid = lax.axis_index("s") * NC + lax.axis_index("c")
        f0 = wid * 2
        pairs = [(users_h, [(0, umf_h), (2, umlp_h)]),
                 (items_h, [(1, imf_h), (3, imlp_h)])]
        pend = [None, None]
        for idx_h, tbls in pairs:
            pltpu.sync_copy(idx_h, idx_v)
            for t, tbl in tbls:
                for f in range(2):
                    col = f0 + f
                    pltpu.sync_copy(tbl.at[col], colbuf)
                    for q in range(4):
                        slot = q % 2
                        outq = outq_a if slot == 0 else outq_b
                        sem = sem_a if slot == 0 else sem_b
                        if pend[slot] is not None:
                            pend[slot].wait()
                            pend[slot] = None

                        def gather_body(v, carry, q=q, outq=outq):
                            base = q * QTR + v * 128
                            for k in range(8):
                                iv = idx_v[pl.ds(base + k * 16, 16)]
                                outq[pl.ds(v * 128 + k * 16, 16)] = (
                                    plsc.load_gather(colbuf, [iv]))
                            return carry

                        lax.fori_loop(0, QTR // 128, gather_body, 0)
                        pend[slot] = pltpu.async_copy(
                            outq,
                            out_o.at[t * D + col, pl.ds(q * QTR, QTR)],
                            sem)
        for slot in range(2):
            if pend[slot] is not None:
                pend[slot].wait()

    return sc_kernel(users, items, umf_t, imf_t, umlp_t, imlp_t)


def _tc_body(g_r, w1at_r, w1bt_r, b1_r, w2ft_r, b2f_r,
             wmf_r, wmlp_r, c0_r, out_r):
    g = g_r[:]
    umf_g = g[0:D]
    imf_g = g[D:2 * D]
    ug_g = g[2 * D:3 * D]
    ig_g = g[3 * D:4 * D]
    h1 = jnp.dot(w1at_r[:], ug_g, preferred_element_type=jnp.float32)
    h1 = h1 + jnp.dot(w1bt_r[:], ig_g, preferred_element_type=jnp.float32)
    h1 = jnp.maximum(h1 + b1_r[:], 0.0)
    h2 = jnp.dot(w2ft_r[:], h1, preferred_element_type=jnp.float32) + b2f_r[:]
    h2 = jnp.maximum(h2, 0.0)
    prod = umf_g * imf_g
    mf = jnp.dot(wmf_r[:], prod, preferred_element_type=jnp.float32)
    ml = jnp.dot(wmlp_r[:], h2, preferred_element_type=jnp.float32)
    out_r[:] = mf + ml + c0_r[0, 0]


def _tc_head(g, w1at, w1bt, b1c, w2ft, b2fc, wmf_row, wmlp_row, c0):
    def bs_full(shape):
        return pl.BlockSpec(shape, lambda i: (0,) * len(shape))

    return pl.pallas_call(
        _tc_body,
        grid=(B // BKC,),
        in_specs=[
            pl.BlockSpec((4 * D, BKC), lambda i: (0, i)),
            bs_full((D, H1)), bs_full((D, H1)), bs_full((H1, 1)),
            bs_full((H2, H1)), bs_full((H2, 1)),
            bs_full((1, D)), bs_full((1, H2)), bs_full((1, 1)),
        ],
        out_specs=pl.BlockSpec((1, BKC), lambda i: (0, i)),
        out_shape=jax.ShapeDtypeStruct((1, B), jnp.float32),
    )(g, w1at, w1bt, b1c, w2ft, b2fc, wmf_row, wmlp_row, c0)


def kernel(users, items, user_mf, item_mf, user_mlp, item_mlp,
           W1, b1, g1, be1, m1, v1, W2, b2, g2, be2, m2, v2, Wp, bp):
    users = users.astype(jnp.int32)
    items = items.astype(jnp.int32)

    g = _sc_gather(users, items,
                   user_mf.T, item_mf.T, user_mlp.T, item_mlp.T)

    s1 = g1 / jnp.sqrt(v1 + EPS)
    t1 = be1 - m1 * s1
    s2 = g2 / jnp.sqrt(v2 + EPS)
    t2 = be2 - m2 * s2
    w1at = W1[:D].T
    w1bt = W1[D:].T
    w2ft = (s1[:, None] * W2).T
    b2f = b2 + t1 @ W2
    wmf = Wp[:D, 0]
    wmlp = s2 * Wp[D:, 0]
    c0 = t2 @ Wp[D:, 0] + bp[0]

    out = _tc_head(g, w1at, w1bt, b1.reshape(H1, 1),
                   w2ft, b2f.reshape(H2, 1),
                   wmf.reshape(1, D), wmlp.reshape(1, H2),
                   c0.reshape(1, 1))
    return out[0]
